# baseline (speedup 1.0000x reference)
import functools

import jax
import jax.numpy as jnp
from jax import lax
from jax.experimental import pallas as pl
from jax.experimental.pallas import tpu as pltpu
from jax.experimental.pallas import tpu_sc as plsc

_AFF_DIM = 64
_OUT_DIM = 128
_N_AFF = 16
_N_MAT = 16


def _lut_body(aff_ref, mat_ref, wa_ref, wm_ref, b_ref, lut_ref):
    aff_proj = lax.dot_general(
        aff_ref[...], wa_ref[...], (((1,), (1,)), ((), ())),
        preferred_element_type=jnp.float32)
    mat_proj = lax.dot_general(
        mat_ref[...], wm_ref[...], (((1,), (1,)), ((), ())),
        preferred_element_type=jnp.float32)
    s = aff_proj[:, None, :] + mat_proj[None, :, :] + b_ref[...][None, :, :]
    lut_ref[...] = jnp.maximum(s, 0.0)


_lut_call = pl.pallas_call(
    _lut_body,
    out_shape=jax.ShapeDtypeStruct((_N_AFF, _N_MAT, _OUT_DIM), jnp.float32),
)

_NC = 2
_NS = 16
_NW = _NC * _NS
_B = 16384
_BPW = _B // _NW
_CH = 128
_NCH = _BPW // _CH

_LANES = 16
_LUT_ROWS = _N_AFF * _N_MAT


def _vbroadcast(vec, lane):
    idx = jnp.full((_LANES, 1), lane, jnp.int32)
    dnums = lax.GatherDimensionNumbers(
        offset_dims=(), collapsed_slice_dims=(0,), start_index_map=(0,))
    return lax.gather(vec, idx, dnums, (1,),
                      mode=lax.GatherScatterMode.PROMISE_IN_BOUNDS)


@functools.lru_cache(maxsize=1)
def _make_gather_kernel():
    mesh = plsc.VectorSubcoreMesh(core_axis_name="c", subcore_axis_name="s",
                                  num_cores=_NC, num_subcores=_NS)

    @functools.partial(
        pl.kernel,
        mesh=mesh,
        compiler_params=pltpu.CompilerParams(needs_layout_passes=False),
        out_type=jax.ShapeDtypeStruct((_B * _OUT_DIM,), jnp.float32),
        scratch_types=[
            pltpu.VMEM((_BPW,), jnp.int32),
            pltpu.VMEM((_BPW,), jnp.int32),
            pltpu.VMEM((_LUT_ROWS, _OUT_DIM), jnp.float32),
            pltpu.VMEM((_BPW * _OUT_DIM,), jnp.float32),
            pltpu.SemaphoreType.DMA,
            pltpu.SemaphoreType.DMA,
        ],
    )
    def gather_kernel(aff_hbm, mat_hbm, lut_hbm, out_hbm,
                      aidx_v, midx_v, lut_v, rows_v, lsem, wsem):
        wid = lax.axis_index("s") * _NC + lax.axis_index("c")
        base = wid * _BPW
        lut_copy = pltpu.async_copy(lut_hbm, lut_v, lsem)
        pltpu.sync_copy(aff_hbm.at[pl.ds(base, _BPW)], aidx_v)
        pltpu.sync_copy(mat_hbm.at[pl.ds(base, _BPW)], midx_v)
        lut_copy.wait()
        iota = lax.iota(jnp.int32, _LANES)
        writes = []

        def do_group(g):
            src = pl.ds(g * _LANES, _LANES)
            c16 = aidx_v[src] * _N_MAT + midx_v[src]
            outoff = g * (_LANES * _OUT_DIM)
            for b in range(_LANES):
                row16 = _vbroadcast(c16, b)
                for c in range(_OUT_DIM // _LANES):
                    v = plsc.load_gather(lut_v, [row16, iota + (c * _LANES)])
                    rows_v[pl.ds(outoff + b * _OUT_DIM + c * _LANES, _LANES)] = v

        for j in range(_NCH):
            lax.fori_loop(j * (_CH // _LANES), (j + 1) * (_CH // _LANES),
                          lambda g, _: (do_group(g), 0)[1], 0)
            writes.append(
                pltpu.async_copy(
                    rows_v.at[pl.ds(j * _CH * _OUT_DIM, _CH * _OUT_DIM)],
                    out_hbm.at[pl.ds((base + j * _CH) * _OUT_DIM,
                                     _CH * _OUT_DIM)], wsem))
        for c in writes:
            c.wait()

    return gather_kernel


def kernel(aff_idx, mat_idx, aff_table, mat_table, W, b):
    lut3 = _lut_call(aff_table, mat_table,
                     W[:, :_AFF_DIM], W[:, _AFF_DIM:], b.reshape(1, _OUT_DIM))
    lut = lut3.reshape(_N_AFF * _N_MAT, _OUT_DIM)
    out = _make_gather_kernel()(aff_idx.astype(jnp.int32),
                                mat_idx.astype(jnp.int32), lut)
    return out.reshape(_B, _OUT_DIM)


# speedup vs baseline: 1.1937x; 1.1937x over previous
import functools

import jax
import jax.numpy as jnp
from jax import lax
from jax.experimental import pallas as pl
from jax.experimental.pallas import tpu as pltpu
from jax.experimental.pallas import tpu_sc as plsc

_AFF_DIM = 64
_OUT_DIM = 128
_N_AFF = 16
_N_MAT = 16


def _lut_body(aff_ref, mat_ref, w_ref, b_ref, lut_ref):
    w = w_ref[...]
    aff_proj = lax.dot_general(
        aff_ref[...], w[:, :_AFF_DIM], (((1,), (1,)), ((), ())),
        preferred_element_type=jnp.float32)
    mat_proj = lax.dot_general(
        mat_ref[...], w[:, _AFF_DIM:], (((1,), (1,)), ((), ())),
        preferred_element_type=jnp.float32)
    s = aff_proj[:, None, :] + mat_proj[None, :, :] + b_ref[...][None, :, :]
    lut_ref[...] = jnp.maximum(s, 0.0)


_lut_call = pl.pallas_call(
    _lut_body,
    out_shape=jax.ShapeDtypeStruct((_N_AFF, _N_MAT, _OUT_DIM), jnp.float32),
)

_NC = 2
_NS = 16
_NW = _NC * _NS
_B = 16384
_BPW = _B // _NW
_CH = 128
_NCH = _BPW // _CH

_LANES = 16
_LUT_ROWS = _N_AFF * _N_MAT


def _vbroadcast(vec, lane):
    idx = jnp.full((_LANES, 1), lane, jnp.int32)
    dnums = lax.GatherDimensionNumbers(
        offset_dims=(), collapsed_slice_dims=(0,), start_index_map=(0,))
    return lax.gather(vec, idx, dnums, (1,),
                      mode=lax.GatherScatterMode.PROMISE_IN_BOUNDS)


@functools.lru_cache(maxsize=1)
def _make_gather_kernel():
    mesh = plsc.VectorSubcoreMesh(core_axis_name="c", subcore_axis_name="s",
                                  num_cores=_NC, num_subcores=_NS)

    @functools.partial(
        pl.kernel,
        mesh=mesh,
        out_type=jax.ShapeDtypeStruct((_B, _OUT_DIM), jnp.float32),
        scratch_types=[
            pltpu.VMEM((_BPW,), jnp.int32),
            pltpu.VMEM((_BPW,), jnp.int32),
            pltpu.VMEM((_NCH, _CH), jnp.int32),
            pltpu.VMEM((_BPW, _OUT_DIM), jnp.float32),
            pltpu.SemaphoreType.DMA,
            pltpu.SemaphoreType.DMA,
        ],
    )
    def gather_kernel(aff_hbm, mat_hbm, lut_hbm, out_hbm,
                      aidx_v, midx_v, cidx_v, rows_v, gsem, wsem):
        wid = lax.axis_index("s") * _NC + lax.axis_index("c")
        base = wid * _BPW
        pltpu.sync_copy(aff_hbm.at[pl.ds(base, _BPW)], aidx_v)
        pltpu.sync_copy(mat_hbm.at[pl.ds(base, _BPW)], midx_v)
        for j in range(_NCH):
            for i in range(_CH // 16):
                src = pl.ds(j * _CH + i * 16, 16)
                cidx_v[j, pl.ds(i * 16, 16)] = aidx_v[src] * _N_MAT + midx_v[src]
        gathers = [
            pltpu.async_copy(lut_hbm.at[cidx_v.at[j]],
                             rows_v.at[pl.ds(j * _CH, _CH)], gsem)
            for j in range(_NCH)
        ]
        half = _BPW // 2
        for g in gathers[:_NCH // 2]:
            g.wait()
        w0 = pltpu.async_copy(rows_v.at[pl.ds(0, half)],
                              out_hbm.at[pl.ds(base, half)], wsem)
        for g in gathers[_NCH // 2:]:
            g.wait()
        w1 = pltpu.async_copy(rows_v.at[pl.ds(half, half)],
                              out_hbm.at[pl.ds(base + half, half)], wsem)
        w0.wait()
        w1.wait()

    return gather_kernel


def kernel(aff_idx, mat_idx, aff_table, mat_table, W, b):
    lut3 = _lut_call(aff_table, mat_table, W, b.reshape(1, _OUT_DIM))
    lut = lut3.reshape(_N_AFF * _N_MAT, _OUT_DIM)
    return _make_gather_kernel()(aff_idx.astype(jnp.int32),
                                 mat_idx.astype(jnp.int32), lut)


# speedup vs baseline: 1.6806x; 1.4079x over previous
import functools

import jax
import jax.numpy as jnp
from jax import lax
from jax.experimental import pallas as pl
from jax.experimental.pallas import tpu as pltpu
from jax.experimental.pallas import tpu_sc as plsc

_AFF_DIM = 64
_OUT_DIM = 128
_N_AFF = 16
_N_MAT = 16


def _lut_body(aff_ref, mat_ref, w_ref, b_ref, lut_ref):
    w = w_ref[...]
    aff_proj = lax.dot_general(
        aff_ref[...], w[:, :_AFF_DIM], (((1,), (1,)), ((), ())),
        preferred_element_type=jnp.float32)
    mat_proj = lax.dot_general(
        mat_ref[...], w[:, _AFF_DIM:], (((1,), (1,)), ((), ())),
        preferred_element_type=jnp.float32)
    s = aff_proj[:, None, :] + mat_proj[None, :, :] + b_ref[...][None, :, :]
    lut_ref[...] = jnp.maximum(s, 0.0)


_lut_call = pl.pallas_call(
    _lut_body,
    out_shape=jax.ShapeDtypeStruct((_N_AFF, _N_MAT, _OUT_DIM), jnp.float32),
)

_NC = 2
_NS = 16
_NW = _NC * _NS
_B = 16384
_BPW = _B // _NW
_CH = 128
_NCH = _BPW // _CH

_LANES = 16
_LUT_ROWS = _N_AFF * _N_MAT


def _vbroadcast(vec, lane):
    idx = jnp.full((_LANES, 1), lane, jnp.int32)
    dnums = lax.GatherDimensionNumbers(
        offset_dims=(), collapsed_slice_dims=(0,), start_index_map=(0,))
    return lax.gather(vec, idx, dnums, (1,),
                      mode=lax.GatherScatterMode.PROMISE_IN_BOUNDS)


@functools.lru_cache(maxsize=1)
def _make_gather_kernel():
    mesh = plsc.VectorSubcoreMesh(core_axis_name="c", subcore_axis_name="s",
                                  num_cores=_NC, num_subcores=_NS)

    @functools.partial(
        pl.kernel,
        mesh=mesh,
        out_type=jax.ShapeDtypeStruct((_B, _OUT_DIM), jnp.float32),
        scratch_types=[
            pltpu.VMEM((_BPW,), jnp.int32),
            pltpu.VMEM((_BPW,), jnp.int32),
            pltpu.VMEM((_NCH, _CH), jnp.int32),
            pltpu.VMEM((_BPW, _OUT_DIM), jnp.float32),
            pltpu.VMEM_SHARED((_LUT_ROWS, _OUT_DIM), jnp.float32),
            pltpu.SemaphoreType.DMA,
            pltpu.SemaphoreType.DMA,
        ],
    )
    def gather_kernel(aff_hbm, mat_hbm, lut_hbm, out_hbm,
                      aidx_v, midx_v, cidx_v, rows_v, lut_sh, gsem, wsem):
        wid = lax.axis_index("s") * _NC + lax.axis_index("c")
        base = wid * _BPW

        @pl.when(lax.axis_index("s") == 0)
        def _stage_lut():
            pltpu.sync_copy(lut_hbm, lut_sh)

        pltpu.sync_copy(aff_hbm.at[pl.ds(base, _BPW)], aidx_v)
        pltpu.sync_copy(mat_hbm.at[pl.ds(base, _BPW)], midx_v)
        for j in range(_NCH):
            for i in range(_CH // 16):
                src = pl.ds(j * _CH + i * 16, 16)
                cidx_v[j, pl.ds(i * 16, 16)] = aidx_v[src] * _N_MAT + midx_v[src]
        plsc.subcore_barrier()
        gathers = [
            pltpu.async_copy(lut_sh.at[cidx_v.at[j]],
                             rows_v.at[pl.ds(j * _CH, _CH)], gsem)
            for j in range(_NCH)
        ]
        half = _BPW // 2
        for g in gathers[:_NCH // 2]:
            g.wait()
        w0 = pltpu.async_copy(rows_v.at[pl.ds(0, half)],
                              out_hbm.at[pl.ds(base, half)], wsem)
        for g in gathers[_NCH // 2:]:
            g.wait()
        w1 = pltpu.async_copy(rows_v.at[pl.ds(half, half)],
                              out_hbm.at[pl.ds(base + half, half)], wsem)
        w0.wait()
        w1.wait()

    return gather_kernel


def kernel(aff_idx, mat_idx, aff_table, mat_table, W, b):
    lut3 = _lut_call(aff_table, mat_table, W, b.reshape(1, _OUT_DIM))
    lut = lut3.reshape(_N_AFF * _N_MAT, _OUT_DIM)
    return _make_gather_kernel()(aff_idx.astype(jnp.int32),
                                 mat_idx.astype(jnp.int32), lut)


# speedup vs baseline: 1.7081x; 1.0164x over previous
import functools

import jax
import jax.numpy as jnp
from jax import lax
from jax.experimental import pallas as pl
from jax.experimental.pallas import tpu as pltpu
from jax.experimental.pallas import tpu_sc as plsc

_AFF_DIM = 64
_OUT_DIM = 128
_N_AFF = 16
_N_MAT = 16


def _lut_body(aff_ref, mat_ref, w_ref, b_ref, lut_ref):
    w = w_ref[...]
    aff_proj = lax.dot_general(
        aff_ref[...], w[:, :_AFF_DIM], (((1,), (1,)), ((), ())),
        preferred_element_type=jnp.float32)
    mat_proj = lax.dot_general(
        mat_ref[...], w[:, _AFF_DIM:], (((1,), (1,)), ((), ())),
        preferred_element_type=jnp.float32)
    s = aff_proj[:, None, :] + mat_proj[None, :, :] + b_ref[...][None, :, :]
    lut_ref[...] = jnp.maximum(s, 0.0)


_lut_call = pl.pallas_call(
    _lut_body,
    out_shape=jax.ShapeDtypeStruct((_N_AFF, _N_MAT, _OUT_DIM), jnp.float32),
)

_NC = 2
_NS = 16
_NW = _NC * _NS
_B = 16384
_BPW = _B // _NW
_CH = 128
_NCH = _BPW // _CH

_LANES = 16
_LUT_ROWS = _N_AFF * _N_MAT


def _vbroadcast(vec, lane):
    idx = jnp.full((_LANES, 1), lane, jnp.int32)
    dnums = lax.GatherDimensionNumbers(
        offset_dims=(), collapsed_slice_dims=(0,), start_index_map=(0,))
    return lax.gather(vec, idx, dnums, (1,),
                      mode=lax.GatherScatterMode.PROMISE_IN_BOUNDS)


@functools.lru_cache(maxsize=1)
def _make_gather_kernel():
    mesh = plsc.VectorSubcoreMesh(core_axis_name="c", subcore_axis_name="s",
                                  num_cores=_NC, num_subcores=_NS)

    @functools.partial(
        pl.kernel,
        mesh=mesh,
        out_type=jax.ShapeDtypeStruct((_B, _OUT_DIM), jnp.float32),
        scratch_types=[
            pltpu.VMEM((_BPW,), jnp.int32),
            pltpu.VMEM((_BPW,), jnp.int32),
            pltpu.VMEM((_NCH, _CH), jnp.int32),
            pltpu.VMEM((_BPW, _OUT_DIM), jnp.float32),
            pltpu.VMEM_SHARED((_LUT_ROWS, _OUT_DIM), jnp.float32),
            pltpu.SemaphoreType.DMA,
            pltpu.SemaphoreType.DMA,
        ],
    )
    def gather_kernel(aff_hbm, mat_hbm, lut_hbm, out_hbm,
                      aidx_v, midx_v, cidx_v, rows_v, lut_sh, gsem, wsem):
        wid = lax.axis_index("s") * _NC + lax.axis_index("c")
        base = wid * _BPW

        @pl.when(lax.axis_index("s") == 0)
        def _stage_lut():
            pltpu.sync_copy(lut_hbm, lut_sh)

        pltpu.sync_copy(aff_hbm.at[pl.ds(base, _BPW)], aidx_v)
        pltpu.sync_copy(mat_hbm.at[pl.ds(base, _BPW)], midx_v)
        for j in range(_NCH):
            for i in range(_CH // 16):
                src = pl.ds(j * _CH + i * 16, 16)
                cidx_v[j, pl.ds(i * 16, 16)] = aidx_v[src] * _N_MAT + midx_v[src]
        plsc.subcore_barrier()
        gathers = [
            pltpu.async_copy(lut_sh.at[cidx_v.at[j]],
                             rows_v.at[pl.ds(j * _CH, _CH)], gsem)
            for j in range(_NCH)
        ]
        writes = []
        for j in range(_NCH):
            gathers[j].wait()
            writes.append(
                pltpu.async_copy(rows_v.at[pl.ds(j * _CH, _CH)],
                                 out_hbm.at[pl.ds(base + j * _CH, _CH)], wsem))
        for w in writes:
            w.wait()

    return gather_kernel


def kernel(aff_idx, mat_idx, aff_table, mat_table, W, b):
    lut3 = _lut_call(aff_table, mat_table, W, b.reshape(1, _OUT_DIM))
    lut = lut3.reshape(_N_AFF * _N_MAT, _OUT_DIM)
    return _make_gather_kernel()(aff_idx.astype(jnp.int32),
                                 mat_idx.astype(jnp.int32), lut)


# speedup vs baseline: 1.7402x; 1.0188x over previous
import functools

import jax
import jax.numpy as jnp
from jax import lax
from jax.experimental import pallas as pl
from jax.experimental.pallas import tpu as pltpu
from jax.experimental.pallas import tpu_sc as plsc

_AFF_DIM = 64
_OUT_DIM = 128
_N_AFF = 16
_N_MAT = 16


def _lut_body(aff_ref, mat_ref, w_ref, b_ref, ai_ref, mi_ref,
              lut_ref, cidx_ref):
    w = w_ref[...]
    aff_proj = lax.dot_general(
        aff_ref[...], w[:, :_AFF_DIM], (((1,), (1,)), ((), ())),
        preferred_element_type=jnp.float32)
    mat_proj = lax.dot_general(
        mat_ref[...], w[:, _AFF_DIM:], (((1,), (1,)), ((), ())),
        preferred_element_type=jnp.float32)
    s = aff_proj[:, None, :] + mat_proj[None, :, :] + b_ref[...][None, :, :]
    lut_ref[...] = jnp.maximum(s, 0.0)
    cidx_ref[...] = ai_ref[...] * _N_MAT + mi_ref[...]


_lut_call = pl.pallas_call(
    _lut_body,
    out_shape=(
        jax.ShapeDtypeStruct((_N_AFF, _N_MAT, _OUT_DIM), jnp.float32),
        jax.ShapeDtypeStruct((128, 128), jnp.int32),
    ),
)

_NC = 2
_NS = 16
_NW = _NC * _NS
_B = 16384
_BPW = _B // _NW
_CH = 128
_NCH = _BPW // _CH

_LANES = 16
_LUT_ROWS = _N_AFF * _N_MAT


def _vbroadcast(vec, lane):
    idx = jnp.full((_LANES, 1), lane, jnp.int32)
    dnums = lax.GatherDimensionNumbers(
        offset_dims=(), collapsed_slice_dims=(0,), start_index_map=(0,))
    return lax.gather(vec, idx, dnums, (1,),
                      mode=lax.GatherScatterMode.PROMISE_IN_BOUNDS)


@functools.lru_cache(maxsize=1)
def _make_gather_kernel():
    mesh = plsc.VectorSubcoreMesh(core_axis_name="c", subcore_axis_name="s",
                                  num_cores=_NC, num_subcores=_NS)

    @functools.partial(
        pl.kernel,
        mesh=mesh,
        out_type=jax.ShapeDtypeStruct((_B, _OUT_DIM), jnp.float32),
        scratch_types=[
            pltpu.VMEM((_NCH, _CH), jnp.int32),
            pltpu.VMEM((_BPW, _OUT_DIM), jnp.float32),
            pltpu.VMEM_SHARED((_LUT_ROWS, _OUT_DIM), jnp.float32),
            pltpu.SemaphoreType.DMA,
            pltpu.SemaphoreType.DMA,
        ],
    )
    def gather_kernel(cidx_hbm, lut_hbm, out_hbm,
                      cidx_v, rows_v, lut_sh, gsem, wsem):
        wid = lax.axis_index("s") * _NC + lax.axis_index("c")
        base = wid * _BPW

        @pl.when(lax.axis_index("s") == 0)
        def _stage_lut():
            pltpu.sync_copy(lut_hbm, lut_sh)

        pltpu.sync_copy(cidx_hbm.at[wid], cidx_v)
        plsc.subcore_barrier()
        gathers = [
            pltpu.async_copy(lut_sh.at[cidx_v.at[j]],
                             rows_v.at[pl.ds(j * _CH, _CH)], gsem)
            for j in range(_NCH)
        ]
        writes = []
        for j in range(_NCH):
            gathers[j].wait()
            writes.append(
                pltpu.async_copy(rows_v.at[pl.ds(j * _CH, _CH)],
                                 out_hbm.at[pl.ds(base + j * _CH, _CH)], wsem))
        for w in writes:
            w.wait()

    return gather_kernel


def kernel(aff_idx, mat_idx, aff_table, mat_table, W, b):
    lut3, cidx2d = _lut_call(aff_table, mat_table, W, b.reshape(1, _OUT_DIM),
                             aff_idx.astype(jnp.int32).reshape(128, 128),
                             mat_idx.astype(jnp.int32).reshape(128, 128))
    lut = lut3.reshape(_N_AFF * _N_MAT, _OUT_DIM)
    cidx = cidx2d.reshape(_NW, _NCH, _CH)
    return _make_gather_kernel()(cidx, lut)


# speedup vs baseline: 1.7451x; 1.0028x over previous
import functools

import jax
import jax.numpy as jnp
from jax import lax
from jax.experimental import pallas as pl
from jax.experimental.pallas import tpu as pltpu
from jax.experimental.pallas import tpu_sc as plsc

_AFF_DIM = 64
_OUT_DIM = 128
_N_AFF = 16
_N_MAT = 16


def _lut_body(aff_ref, mat_ref, w_ref, b_ref, ai_ref, mi_ref,
              lut_ref, cidx_ref):
    w = w_ref[...]
    aff_proj = lax.dot_general(
        aff_ref[...], w[:, :_AFF_DIM], (((1,), (1,)), ((), ())),
        preferred_element_type=jnp.float32)
    mat_proj = lax.dot_general(
        mat_ref[...], w[:, _AFF_DIM:], (((1,), (1,)), ((), ())),
        preferred_element_type=jnp.float32)
    s = aff_proj[:, None, :] + mat_proj[None, :, :] + b_ref[...][None, :, :]
    lut_ref[...] = jnp.maximum(s, 0.0)
    cidx_ref[...] = ai_ref[...] * _N_MAT + mi_ref[...]


_lut_call = pl.pallas_call(
    _lut_body,
    out_shape=(
        jax.ShapeDtypeStruct((_N_AFF, _N_MAT, _OUT_DIM), jnp.float32),
        jax.ShapeDtypeStruct((128, 128), jnp.int32),
    ),
)

_NC = 2
_NS = 16
_NW = _NC * _NS
_B = 16384
_BPW = _B // _NW
_CH = 128
_NCH = _BPW // _CH

_LANES = 16
_LUT_ROWS = _N_AFF * _N_MAT


def _vbroadcast(vec, lane):
    idx = jnp.full((_LANES, 1), lane, jnp.int32)
    dnums = lax.GatherDimensionNumbers(
        offset_dims=(), collapsed_slice_dims=(0,), start_index_map=(0,))
    return lax.gather(vec, idx, dnums, (1,),
                      mode=lax.GatherScatterMode.PROMISE_IN_BOUNDS)


@functools.lru_cache(maxsize=1)
def _make_gather_kernel():
    mesh = plsc.VectorSubcoreMesh(core_axis_name="c", subcore_axis_name="s",
                                  num_cores=_NC, num_subcores=_NS)

    @functools.partial(
        pl.kernel,
        mesh=mesh,
        out_type=jax.ShapeDtypeStruct((_B, _OUT_DIM), jnp.float32),
        scratch_types=[
            pltpu.VMEM((_NCH, _CH), jnp.int32),
            pltpu.VMEM((_BPW, _OUT_DIM), jnp.float32),
            pltpu.VMEM_SHARED((_LUT_ROWS, _OUT_DIM), jnp.float32),
            pltpu.SemaphoreType.DMA,
            pltpu.SemaphoreType.DMA,
        ],
    )
    def gather_kernel(cidx_hbm, lut_hbm, out_hbm,
                      cidx_v, rows_v, lut_sh, gsem, wsem):
        wid = lax.axis_index("s") * _NC + lax.axis_index("c")
        base = wid * _BPW

        sid = lax.axis_index("s")
        rows_per_tile = _LUT_ROWS // _NS
        lrows = pl.ds(sid * rows_per_tile, rows_per_tile)
        pltpu.sync_copy(lut_hbm.at[lrows], lut_sh.at[lrows])
        pltpu.sync_copy(cidx_hbm.at[wid], cidx_v)
        plsc.subcore_barrier()
        gathers = [
            pltpu.async_copy(lut_sh.at[cidx_v.at[j]],
                             rows_v.at[pl.ds(j * _CH, _CH)], gsem)
            for j in range(_NCH)
        ]
        writes = []
        for j in range(_NCH):
            gathers[j].wait()
            writes.append(
                pltpu.async_copy(rows_v.at[pl.ds(j * _CH, _CH)],
                                 out_hbm.at[pl.ds(base + j * _CH, _CH)], wsem))
        for w in writes:
            w.wait()

    return gather_kernel


def kernel(aff_idx, mat_idx, aff_table, mat_table, W, b):
    lut3, cidx2d = _lut_call(aff_table, mat_table, W, b.reshape(1, _OUT_DIM),
                             aff_idx.astype(jnp.int32).reshape(128, 128),
                             mat_idx.astype(jnp.int32).reshape(128, 128))
    lut = lut3.reshape(_N_AFF * _N_MAT, _OUT_DIM)
    cidx = cidx2d.reshape(_NW, _NCH, _CH)
    return _make_gather_kernel()(cidx, lut)
